# baseline (speedup 1.0000x reference)
import functools

import jax
import jax.numpy as jnp
from jax import lax
from jax.experimental import pallas as pl
from jax.experimental.pallas import tpu as pltpu
from jax.experimental.pallas import tpu_sc as plsc

_V = 100000
_D = 32
_B = 4096
_L = 50
_F = 4
_N = _B * _L

_NC = 2
_NS = 16
_NW = _NC * _NS
_PER_W = _N // _NW
_C = 128
_NCH = _PER_W // _C

_MESH = plsc.VectorSubcoreMesh(core_axis_name="c", subcore_axis_name="s")
_SCRATCH = [
    pltpu.VMEM((_NCH, _C), jnp.int32),
    pltpu.VMEM((_C, _D), jnp.float32),
    pltpu.VMEM((_C, _D), jnp.float32),
    pltpu.SemaphoreType.DMA,
    pltpu.SemaphoreType.DMA,
    pltpu.SemaphoreType.DMA,
    pltpu.SemaphoreType.DMA,
]


def _field_pipeline(col, ih, eh, out, idx_v, rows_a, rows_b,
                    gsem_a, gsem_b, wsem_a, wsem_b):
    wid = lax.axis_index("s") * _NC + lax.axis_index("c")
    base = wid * _PER_W

    pltpu.sync_copy(ih.at[pl.ds(wid * _NCH, _NCH)], idx_v)

    def fire_gather(ci, rows, gsem):
        pltpu.async_copy(eh.at[idx_v.at[ci]], rows, gsem)

    def wait_gather(rows, gsem):
        pltpu.make_async_copy(eh.at[pl.ds(0, _C)], rows, gsem).wait()

    def fire_write(ci, rows, wsem):
        off = base + ci * _C
        pltpu.async_copy(rows, out.at[pl.ds(off, _C), pl.ds(col, _D)], wsem)

    def wait_write(rows, wsem):
        pltpu.make_async_copy(rows, out.at[pl.ds(0, _C), pl.ds(col, _D)],
                              wsem).wait()

    fire_gather(0, rows_a, gsem_a)
    fire_gather(1, rows_b, gsem_b)

    def body(j, _):
        ca = 2 * j
        wait_gather(rows_a, gsem_a)
        fire_write(ca, rows_a, wsem_a)
        wait_gather(rows_b, gsem_b)
        fire_write(ca + 1, rows_b, wsem_b)
        wait_write(rows_a, wsem_a)

        @pl.when(ca + 2 < _NCH)
        def _():
            fire_gather(ca + 2, rows_a, gsem_a)

        wait_write(rows_b, wsem_b)

        @pl.when(ca + 3 < _NCH)
        def _():
            fire_gather(ca + 3, rows_b, gsem_b)

        return 0

    lax.fori_loop(0, _NCH // 2, body, 0)


@functools.partial(
    pl.kernel,
    out_type=jax.ShapeDtypeStruct((_N, _F * _D), jnp.float32),
    mesh=_MESH,
    compiler_params=pltpu.CompilerParams(use_tc_tiling_on_sc=False),
    scratch_types=_SCRATCH,
)
def _gather_first(ih, eh, out, *scratch):
    _field_pipeline(0, ih, eh, out, *scratch)


def _make_ref_kernel(col):
    @functools.partial(
        pl.kernel,
        out_type=(),
        mesh=_MESH,
        compiler_params=pltpu.CompilerParams(use_tc_tiling_on_sc=False),
        scratch_types=_SCRATCH,
    )
    def k(ih, eh, out, *scratch):
        _field_pipeline(col, ih, eh, out, *scratch)

    return k


_gather_rest = [_make_ref_kernel(f * _D) for f in range(1, _F)]


def kernel(idx_cat0, idx_cat1, idx_cat2, idx_cat3,
           emb_cat0, emb_cat1, emb_cat2, emb_cat3):
    idxs = [i.T.reshape(_NW * _NCH, _C).astype(jnp.int32)
            for i in (idx_cat0, idx_cat1, idx_cat2, idx_cat3)]
    embs = [emb_cat0, emb_cat1, emb_cat2, emb_cat3]
    out0 = _gather_first(idxs[0], embs[0])
    ref = jax.new_ref(out0)
    for f in range(1, _F):
        _gather_rest[f - 1](idxs[f], embs[f], ref)
    out = ref[...]
    return out.reshape(_L, _B, _F * _D).transpose(1, 0, 2)


# speedup vs baseline: 1.0887x; 1.0887x over previous
import functools

import jax
import jax.numpy as jnp
from jax import lax
from jax.experimental import pallas as pl
from jax.experimental.pallas import tpu as pltpu
from jax.experimental.pallas import tpu_sc as plsc

_V = 100000
_D = 32
_B = 4096
_L = 50
_F = 4
_N = _B * _L

_NC = 2
_NS = 16
_NW = _NC * _NS
_PER_W = _N // _NW
_C = 128
_NCH = _PER_W // _C

_MESH = plsc.VectorSubcoreMesh(core_axis_name="c", subcore_axis_name="s")
_NBUF = 4
_SCRATCH = (
    [pltpu.VMEM((_NCH, _C), jnp.int32)]
    + [pltpu.VMEM((_C, _D), jnp.float32) for _ in range(_NBUF)]
    + [pltpu.SemaphoreType.DMA for _ in range(2 * _NBUF)]
)


def _field_pipeline(col, ih, eh, out, idx_v, *rest):
    bufs = rest[:_NBUF]
    gsems = rest[_NBUF:2 * _NBUF]
    wsems = rest[2 * _NBUF:3 * _NBUF]
    wid = lax.axis_index("s") * _NC + lax.axis_index("c")
    base = wid * _PER_W

    pltpu.sync_copy(ih.at[pl.ds(wid * _NCH, _NCH)], idx_v)

    def fire_gather(ci, k):
        pltpu.async_copy(eh.at[idx_v.at[ci]], bufs[k], gsems[k])

    def wait_gather(k):
        pltpu.make_async_copy(eh.at[pl.ds(0, _C)], bufs[k], gsems[k]).wait()

    def fire_write(ci, k):
        off = base + ci * _C
        pltpu.async_copy(bufs[k], out.at[pl.ds(off, _C), pl.ds(col, _D)],
                         wsems[k])

    def wait_write(k):
        pltpu.make_async_copy(bufs[k], out.at[pl.ds(0, _C), pl.ds(col, _D)],
                              wsems[k]).wait()

    for k in range(_NBUF):
        fire_gather(k, k)

    def body(j, _):
        c0 = _NBUF * j
        for k in range(_NBUF):
            wait_gather(k)
            fire_write(c0 + k, k)
        for k in range(_NBUF):
            wait_write(k)

            @pl.when(c0 + k + _NBUF < _NCH)
            def _(k=k):
                fire_gather(c0 + k + _NBUF, k)

        return 0

    lax.fori_loop(0, _NCH // _NBUF, body, 0)

    for k in range(_NCH % _NBUF):
        wait_gather(k)
        fire_write((_NCH // _NBUF) * _NBUF + k, k)
    for k in range(_NCH % _NBUF):
        wait_write(k)


@functools.partial(
    pl.kernel,
    out_type=jax.ShapeDtypeStruct((_N, _F * _D), jnp.float32),
    mesh=_MESH,
    compiler_params=pltpu.CompilerParams(use_tc_tiling_on_sc=False),
    scratch_types=_SCRATCH,
)
def _gather_first(ih, eh, out, *scratch):
    _field_pipeline(0, ih, eh, out, *scratch)


def _make_ref_kernel(col):
    @functools.partial(
        pl.kernel,
        out_type=(),
        mesh=_MESH,
        compiler_params=pltpu.CompilerParams(use_tc_tiling_on_sc=False),
        scratch_types=_SCRATCH,
    )
    def k(ih, eh, out, *scratch):
        _field_pipeline(col, ih, eh, out, *scratch)

    return k


_gather_rest = [_make_ref_kernel(f * _D) for f in range(1, _F)]


def kernel(idx_cat0, idx_cat1, idx_cat2, idx_cat3,
           emb_cat0, emb_cat1, emb_cat2, emb_cat3):
    idxs = [i.T.reshape(_NW * _NCH, _C).astype(jnp.int32)
            for i in (idx_cat0, idx_cat1, idx_cat2, idx_cat3)]
    embs = [emb_cat0, emb_cat1, emb_cat2, emb_cat3]
    out0 = _gather_first(idxs[0], embs[0])
    ref = jax.new_ref(out0)
    for f in range(1, _F):
        _gather_rest[f - 1](idxs[f], embs[f], ref)
    out = ref[...]
    return out.reshape(_L, _B, _F * _D).transpose(1, 0, 2)
